# baseline (speedup 1.0000x reference)
import functools

import jax
import jax.numpy as jnp
from jax import lax
from jax.experimental import pallas as pl
from jax.experimental.pallas import tpu as pltpu
from jax.experimental.pallas import tpu_sc as plsc

import numpy as np

_NEG = np.float32(-1e9)
_BLK = 256


def _sc_gather3(t1, tL, t2, idx_a, idx_2):
    B = idx_a.shape[0]
    D = t1.shape[1]
    info = plsc.get_sparse_core_info()
    nw = info.num_cores * info.num_subcores
    b_per_w = B // nw
    mesh = plsc.VectorSubcoreMesh(core_axis_name="c", subcore_axis_name="s")

    @functools.partial(
        pl.kernel,
        mesh=mesh,
        out_type=(
            jax.ShapeDtypeStruct((B, D), jnp.float32),
            jax.ShapeDtypeStruct((B, D), jnp.float32),
            jax.ShapeDtypeStruct((B, D), jnp.float32),
        ),
        scratch_types=[
            pltpu.VMEM((b_per_w,), jnp.int32),
            pltpu.VMEM((b_per_w,), jnp.int32),
            pltpu.VMEM((b_per_w, D), jnp.float32),
            pltpu.VMEM((b_per_w, D), jnp.float32),
            pltpu.VMEM((b_per_w, D), jnp.float32),
            pltpu.SemaphoreType.DMA,
        ],
    )
    def k(t1_hbm, tL_hbm, t2_hbm, ia_hbm, i2_hbm, o1_hbm, oL_hbm, o2_hbm,
          ia_v, i2_v, r1_v, rL_v, r2_v, sem):
        wid = lax.axis_index("s") * info.num_cores + lax.axis_index("c")
        base = wid * b_per_w
        pltpu.sync_copy(ia_hbm.at[pl.ds(base, b_per_w)], ia_v)
        pltpu.sync_copy(i2_hbm.at[pl.ds(base, b_per_w)], i2_v)
        pltpu.async_copy(t1_hbm.at[ia_v], r1_v, sem)
        pltpu.async_copy(tL_hbm.at[ia_v], rL_v, sem)
        pltpu.async_copy(t2_hbm.at[i2_v], r2_v, sem).wait()
        pltpu.make_async_copy(t1_hbm.at[ia_v], r1_v, sem).wait()
        pltpu.make_async_copy(tL_hbm.at[ia_v], rL_v, sem).wait()
        pltpu.sync_copy(r1_v, o1_hbm.at[pl.ds(base, b_per_w)])
        pltpu.sync_copy(rL_v, oL_hbm.at[pl.ds(base, b_per_w)])
        pltpu.sync_copy(r2_v, o2_hbm.at[pl.ds(base, b_per_w)])

    return k(t1, tL, t2, idx_a, idx_2)


def _combine_body(enc_ref, wcat_ref, at_ref, o1_ref, loc_ref, o2_ref,
                  ia_ref, i2_ref,
                  atm_ref, m1_ref, mL_ref, m2_ref,
                  bat_ref, bo1_ref, bloc_ref, bo2_ref,
                  t1_ref, tLa_ref, tLo_ref, t2a_ref, t2o_ref, t2l_ref,
                  out_ref):
    f32 = jnp.float32
    S = jnp.dot(enc_ref[...], wcat_ref[...], preferred_element_type=f32)
    n = S.shape[0]
    at = at_ref[:, 0]
    o1 = o1_ref[:, 0]
    loc = loc_ref[:, 0]
    o2 = o2_ref[:, 0]
    pa = (ia_ref[:, 0] % 2) * 64
    p2 = (i2_ref[:, 0] % 2) * 64
    ioA = lax.broadcasted_iota(jnp.int32, (n, 8), 1)
    io64 = lax.broadcasted_iota(jnp.int32, (n, 64), 1)
    io128 = lax.broadcasted_iota(jnp.int32, (n, 128), 1)
    oh_at = (ioA == at[:, None]).astype(f32)
    oh_o1 = (io64 == o1[:, None]).astype(f32)
    oh_loc = (io64 == loc[:, None]).astype(f32)
    oh_o2 = (io64 == o2[:, None]).astype(f32)

    row_at = S[:, 0:8] + bat_ref[...]
    s_at = jnp.sum(oh_at * row_at, axis=1)
    m_at = jnp.sum(oh_at * atm_ref[...], axis=1)
    q = jnp.where(m_at > 0.0, s_at, _NEG)

    row1 = (S[:, 8:72]
            + jnp.dot(oh_at, t1_ref[...], preferred_element_type=f32)
            + bo1_ref[...])
    s1 = jnp.sum(oh_o1 * row1, axis=1)
    m1 = jnp.sum(jnp.where(io128 == (pa + o1)[:, None], m1_ref[...], 0.0),
                 axis=1)
    q = q + jnp.where(m1 > 0.0, s1, _NEG)

    rowL = (S[:, 72:136]
            + jnp.dot(oh_at, tLa_ref[...], preferred_element_type=f32)
            + jnp.dot(oh_o1, tLo_ref[...], preferred_element_type=f32)
            + bloc_ref[...])
    sL = jnp.sum(oh_loc * rowL, axis=1)
    mL = jnp.sum(jnp.where(io128 == (pa + loc)[:, None], mL_ref[...], 0.0),
                 axis=1)
    q = q + jnp.where(mL > 0.0, sL, _NEG)

    row2 = (S[:, 136:200]
            + jnp.dot(oh_at, t2a_ref[...], preferred_element_type=f32)
            + jnp.dot(oh_o1, t2o_ref[...], preferred_element_type=f32)
            + jnp.dot(oh_loc, t2l_ref[...], preferred_element_type=f32)
            + bo2_ref[...])
    s2 = jnp.sum(oh_o2 * row2, axis=1)
    m2 = jnp.sum(jnp.where(io128 == (p2 + o2)[:, None], m2_ref[...], 0.0),
                 axis=1)
    q = q + jnp.where(m2 > 0.0, s2, _NEG)

    out_ref[...] = q[:, None]


def _tc_combine(enc, wcat, at, o1, loc, o2, ia, i2, atm, m1r, mLr, m2r,
                b_at, b_o1, b_loc, b_o2, T1, TLa, TLo, T2a, T2o, T2l):
    B, H = enc.shape

    def rows(i):
        return (i, 0)

    def full(i):
        return (0, 0)

    def fixed(a):
        return pl.BlockSpec(a.shape, full)

    return pl.pallas_call(
        _combine_body,
        grid=(B // _BLK,),
        in_specs=[
            pl.BlockSpec((_BLK, H), rows),
            fixed(wcat),
            pl.BlockSpec((_BLK, 1), rows),
            pl.BlockSpec((_BLK, 1), rows),
            pl.BlockSpec((_BLK, 1), rows),
            pl.BlockSpec((_BLK, 1), rows),
            pl.BlockSpec((_BLK, 1), rows),
            pl.BlockSpec((_BLK, 1), rows),
            pl.BlockSpec((_BLK, atm.shape[1]), rows),
            pl.BlockSpec((_BLK, m1r.shape[1]), rows),
            pl.BlockSpec((_BLK, mLr.shape[1]), rows),
            pl.BlockSpec((_BLK, m2r.shape[1]), rows),
            fixed(b_at), fixed(b_o1), fixed(b_loc), fixed(b_o2),
            fixed(T1), fixed(TLa), fixed(TLo), fixed(T2a), fixed(T2o),
            fixed(T2l),
        ],
        out_specs=pl.BlockSpec((_BLK, 1), rows),
        out_shape=jax.ShapeDtypeStruct((B, 1), jnp.float32),
    )(enc, wcat, at, o1, loc, o2, ia, i2, atm, m1r, mLr, m2r,
      b_at, b_o1, b_loc, b_o2, T1, TLa, TLo, T2a, T2o, T2l)


def kernel(encoded, action_types, object1, location, object2,
           action_type_masks, object1_masks, location_masks, object2_masks,
           W_at, b_at, W_o1, b_o1, W_loc, b_loc, W_o2, b_o2):
    B, H = encoded.shape
    A = action_type_masks.shape[1]
    O = object1_masks.shape[2]
    L = location_masks.shape[2]

    wcat = jnp.concatenate([W_at, W_o1[:H], W_loc[:H], W_o2[:H]], axis=1)
    pad = (-wcat.shape[1]) % 128
    wcat = jnp.pad(wcat, ((0, 0), (0, pad)))

    at = action_types[:, 0].astype(jnp.int32)
    o1i = object1[:, 0].astype(jnp.int32)
    idx_a = jnp.arange(B, dtype=jnp.int32) * A + at
    idx_2 = idx_a * O + o1i

    m1r, mLr, m2r = _sc_gather3(
        object1_masks.reshape(B * A // 2, 2 * O),
        location_masks.reshape(B * A // 2, 2 * L),
        object2_masks.reshape(B * A * O // 2, 2 * O),
        idx_a // 2, idx_2 // 2)

    return _tc_combine(
        encoded, wcat,
        action_types.astype(jnp.int32), object1.astype(jnp.int32),
        location.astype(jnp.int32), object2.astype(jnp.int32),
        idx_a[:, None], idx_2[:, None],
        action_type_masks, m1r, mLr, m2r,
        b_at.reshape(1, A), b_o1.reshape(1, O), b_loc.reshape(1, L),
        b_o2.reshape(1, O),
        W_o1[H:], W_loc[H:H + A], W_loc[H + A:],
        W_o2[H:H + A], W_o2[H + A:H + A + O], W_o2[H + A + O:])


# speedup vs baseline: 5.9237x; 5.9237x over previous
import functools

import jax
import jax.numpy as jnp
import numpy as np
from jax import lax
from jax.experimental import pallas as pl
from jax.experimental.pallas import tpu as pltpu
from jax.experimental.pallas import tpu_sc as plsc

_NEG = np.float32(-1e9)
_BLK = 256


def _sc_gather_masks(tA, t1, tL, t2, jA, j1, jL, j2):
    B = jA.shape[0]
    info = plsc.get_sparse_core_info()
    nw = info.num_cores * info.num_subcores
    bw = B // nw
    mesh = plsc.VectorSubcoreMesh(core_axis_name="c", subcore_axis_name="s")
    f32 = jnp.float32

    @functools.partial(
        pl.kernel,
        mesh=mesh,
        out_type=tuple(jax.ShapeDtypeStruct((B,), f32) for _ in range(4)),
        scratch_types=[
            pltpu.VMEM((bw,), jnp.int32),
            pltpu.VMEM((bw, B), f32),
            pltpu.VMEM((bw,), f32),
            pltpu.SemaphoreType.DMA,
        ],
    )
    def k(tA_h, t1_h, tL_h, t2_h, jA_h, j1_h, jL_h, j2_h,
          oA_h, o1_h, oL_h, o2_h, idx_v, buf_v, out_v, sem):
        wid = lax.axis_index("s") * info.num_cores + lax.axis_index("c")
        base = wid * bw
        for t_h, j_h, o_h in ((tA_h, jA_h, oA_h), (t1_h, j1_h, o1_h),
                              (tL_h, jL_h, oL_h), (t2_h, j2_h, o2_h)):
            pltpu.sync_copy(j_h.at[pl.ds(base, bw)], idx_v)
            pltpu.async_copy(t_h.at[idx_v], buf_v, sem).wait()
            it = lax.iota(jnp.int32, 16)
            for g in range(bw // 16):
                acc = jnp.zeros((16,), f32)
                for r in range(16):
                    row = buf_v[g * 16 + r, pl.ds(base + g * 16, 16)]
                    acc = jnp.where(it == r, row, acc)
                out_v[pl.ds(g * 16, 16)] = acc
            pltpu.sync_copy(out_v, o_h.at[pl.ds(base, bw)])

    return k(tA, t1, tL, t2, jA, j1, jL, j2)


def _combine_body(enc_ref, wcat_ref, at_ref, o1_ref, loc_ref, o2_ref,
                  mA_ref, m1_ref, mL_ref, m2_ref,
                  bat_ref, bo1_ref, bloc_ref, bo2_ref,
                  t1_ref, tLa_ref, tLo_ref, t2a_ref, t2o_ref, t2l_ref,
                  out_ref):
    f32 = jnp.float32
    S = jnp.dot(enc_ref[...], wcat_ref[...], preferred_element_type=f32)
    n = S.shape[0]
    at = at_ref[:, 0]
    o1 = o1_ref[:, 0]
    loc = loc_ref[:, 0]
    o2 = o2_ref[:, 0]
    ioA = lax.broadcasted_iota(jnp.int32, (n, 8), 1)
    io64 = lax.broadcasted_iota(jnp.int32, (n, 64), 1)
    oh_at = (ioA == at[:, None]).astype(f32)
    oh_o1 = (io64 == o1[:, None]).astype(f32)
    oh_loc = (io64 == loc[:, None]).astype(f32)
    oh_o2 = (io64 == o2[:, None]).astype(f32)

    row_at = S[:, 0:8] + bat_ref[...]
    s_at = jnp.sum(oh_at * row_at, axis=1)
    q = jnp.where(mA_ref[:, 0] > 0.0, s_at, _NEG)

    row1 = (S[:, 8:72]
            + jnp.dot(oh_at, t1_ref[...], preferred_element_type=f32)
            + bo1_ref[...])
    s1 = jnp.sum(oh_o1 * row1, axis=1)
    q = q + jnp.where(m1_ref[:, 0] > 0.0, s1, _NEG)

    rowL = (S[:, 72:136]
            + jnp.dot(oh_at, tLa_ref[...], preferred_element_type=f32)
            + jnp.dot(oh_o1, tLo_ref[...], preferred_element_type=f32)
            + bloc_ref[...])
    sL = jnp.sum(oh_loc * rowL, axis=1)
    q = q + jnp.where(mL_ref[:, 0] > 0.0, sL, _NEG)

    row2 = (S[:, 136:200]
            + jnp.dot(oh_at, t2a_ref[...], preferred_element_type=f32)
            + jnp.dot(oh_o1, t2o_ref[...], preferred_element_type=f32)
            + jnp.dot(oh_loc, t2l_ref[...], preferred_element_type=f32)
            + bo2_ref[...])
    s2 = jnp.sum(oh_o2 * row2, axis=1)
    q = q + jnp.where(m2_ref[:, 0] > 0.0, s2, _NEG)

    out_ref[...] = q[:, None]


def _tc_combine(enc, wcat, at, o1, loc, o2, mA, m1, mL, m2,
                b_at, b_o1, b_loc, b_o2, T1, TLa, TLo, T2a, T2o, T2l):
    B, H = enc.shape

    def rows(i):
        return (i, 0)

    def full(i):
        return (0, 0)

    def fixed(a):
        return pl.BlockSpec(a.shape, full)

    def col(a):
        return pl.BlockSpec((_BLK, 1), rows)

    return pl.pallas_call(
        _combine_body,
        grid=(B // _BLK,),
        in_specs=[
            pl.BlockSpec((_BLK, H), rows),
            fixed(wcat),
            col(at), col(o1), col(loc), col(o2),
            col(mA), col(m1), col(mL), col(m2),
            fixed(b_at), fixed(b_o1), fixed(b_loc), fixed(b_o2),
            fixed(T1), fixed(TLa), fixed(TLo), fixed(T2a), fixed(T2o),
            fixed(T2l),
        ],
        out_specs=pl.BlockSpec((_BLK, 1), rows),
        out_shape=jax.ShapeDtypeStruct((B, 1), jnp.float32),
    )(enc, wcat, at, o1, loc, o2, mA, m1, mL, m2,
      b_at, b_o1, b_loc, b_o2, T1, TLa, TLo, T2a, T2o, T2l)


def kernel(encoded, action_types, object1, location, object2,
           action_type_masks, object1_masks, location_masks, object2_masks,
           W_at, b_at, W_o1, b_o1, W_loc, b_loc, W_o2, b_o2):
    B, H = encoded.shape
    A = action_type_masks.shape[1]
    O = object1_masks.shape[2]
    L = location_masks.shape[2]

    wcat = jnp.concatenate([W_at, W_o1[:H], W_loc[:H], W_o2[:H]], axis=1)
    pad = (-wcat.shape[1]) % 128
    wcat = jnp.pad(wcat, ((0, 0), (0, pad)))

    at = action_types[:, 0].astype(jnp.int32)
    o1i = object1[:, 0].astype(jnp.int32)
    loci = location[:, 0].astype(jnp.int32)
    o2i = object2[:, 0].astype(jnp.int32)

    tA = action_type_masks.transpose(1, 0)
    t1 = object1_masks.transpose(1, 2, 0).reshape(A * O, B)
    tL = location_masks.transpose(1, 2, 0).reshape(A * L, B)
    t2 = object2_masks.transpose(1, 2, 3, 0).reshape(A * O * O, B)

    jA = at
    j1 = at * O + o1i
    jL = at * L + loci
    j2 = (at * O + o1i) * O + o2i

    mA, m1, mL, m2 = _sc_gather_masks(tA, t1, tL, t2, jA, j1, jL, j2)

    return _tc_combine(
        encoded, wcat,
        action_types.astype(jnp.int32), object1.astype(jnp.int32),
        location.astype(jnp.int32), object2.astype(jnp.int32),
        mA[:, None], m1[:, None], mL[:, None], m2[:, None],
        b_at.reshape(1, A), b_o1.reshape(1, O), b_loc.reshape(1, L),
        b_o2.reshape(1, O),
        W_o1[H:], W_loc[H:H + A], W_loc[H + A:],
        W_o2[H:H + A], W_o2[H + A:H + A + O], W_o2[H + A + O:])


# speedup vs baseline: 10.8885x; 1.8381x over previous
import functools

import jax
import jax.numpy as jnp
import numpy as np
from jax import lax
from jax.experimental import pallas as pl
from jax.experimental.pallas import tpu as pltpu
from jax.experimental.pallas import tpu_sc as plsc

_NEG = np.float32(-1e9)
_BLK = 256
_NT = (((1,), (1,)), ((), ()))


def _sc_gather_masks(tA, t1, tL, t2, jA, j1, jL, j2):
    B = jA.shape[0]
    info = plsc.get_sparse_core_info()
    nw = info.num_cores * info.num_subcores
    bw = B // nw
    mesh = plsc.VectorSubcoreMesh(core_axis_name="c", subcore_axis_name="s")
    f32 = jnp.float32

    @functools.partial(
        pl.kernel,
        mesh=mesh,
        out_type=tuple(jax.ShapeDtypeStruct((B,), f32) for _ in range(4)),
        scratch_types=[
            pltpu.VMEM((4, bw), jnp.int32),
            pltpu.VMEM((bw, 128), f32),
            pltpu.VMEM((bw, 128), f32),
            pltpu.VMEM((bw, 128), f32),
            pltpu.VMEM((bw, 128), f32),
            pltpu.VMEM((bw,), f32),
            pltpu.SemaphoreType.DMA,
        ],
    )
    def k(tA_h, t1_h, tL_h, t2_h, jA_h, j1_h, jL_h, j2_h,
          oA_h, o1_h, oL_h, o2_h, idx_v, bA_v, b1_v, bL_v, b2_v, out_v, sem):
        wid = lax.axis_index("s") * info.num_cores + lax.axis_index("c")
        base = wid * bw
        cb = (base // 128) * 128
        co = base - cb
        tabs = ((tA_h, jA_h, oA_h, bA_v), (t1_h, j1_h, o1_h, b1_v),
                (tL_h, jL_h, oL_h, bL_v), (t2_h, j2_h, o2_h, b2_v))
        for i, (_, j_h, _, _) in enumerate(tabs):
            pltpu.sync_copy(j_h.at[pl.ds(base, bw)], idx_v.at[i])
        copies = [
            pltpu.async_copy(t_h.at[idx_v.at[i], pl.ds(cb, 128)], b_v, sem)
            for i, (t_h, _, _, b_v) in enumerate(tabs)
        ]
        for c in copies:
            c.wait()
        it = lax.iota(jnp.int32, 16)
        for t_h, _, o_h, b_v in tabs:
            for g in range(bw // 16):
                acc = jnp.zeros((16,), f32)
                for r in range(16):
                    row = b_v[g * 16 + r, pl.ds(co + g * 16, 16)]
                    acc = jnp.where(it == r, row, acc)
                out_v[pl.ds(g * 16, 16)] = acc
            pltpu.sync_copy(out_v, o_h.at[pl.ds(base, bw)])

    return k(tA, t1, tL, t2, jA, j1, jL, j2)


def _combine_body(enc_ref, watt_ref, w1t_ref, wlt_ref, w2t_ref,
                  at_ref, o1_ref, loc_ref, o2_ref,
                  mA_ref, m1_ref, mL_ref, m2_ref,
                  bat_ref, bo1_ref, bloc_ref, bo2_ref,
                  out_ref):
    f32 = jnp.float32
    H = enc_ref.shape[1]
    A = watt_ref.shape[0]
    O = w1t_ref.shape[0]
    n = enc_ref.shape[0]
    enc = enc_ref[...]

    def nt(x, y):
        return lax.dot_general(x, y, _NT, preferred_element_type=f32)

    at = at_ref[:, 0]
    o1 = o1_ref[:, 0]
    loc = loc_ref[:, 0]
    o2 = o2_ref[:, 0]
    mA = jnp.reshape(mA_ref[...], (n,))
    m1 = jnp.reshape(m1_ref[...], (n,))
    mL = jnp.reshape(mL_ref[...], (n,))
    m2 = jnp.reshape(m2_ref[...], (n,))
    ioA = lax.broadcasted_iota(jnp.int32, (n, A), 1)
    io64 = lax.broadcasted_iota(jnp.int32, (n, O), 1)
    oh_at = (ioA == at[:, None]).astype(f32)
    oh_o1 = (io64 == o1[:, None]).astype(f32)
    oh_loc = (io64 == loc[:, None]).astype(f32)
    oh_o2 = (io64 == o2[:, None]).astype(f32)

    w1t = w1t_ref[...]
    wlt = wlt_ref[...]
    w2t = w2t_ref[...]

    row_at = nt(enc, watt_ref[...]) + bat_ref[...]
    s_at = jnp.sum(oh_at * row_at, axis=1)
    q = jnp.where(mA > 0.0, s_at, _NEG)

    row1 = nt(enc, w1t[:, :H]) + nt(oh_at, w1t[:, H:]) + bo1_ref[...]
    s1 = jnp.sum(oh_o1 * row1, axis=1)
    q = q + jnp.where(m1 > 0.0, s1, _NEG)

    rowL = (nt(enc, wlt[:, :H]) + nt(oh_at, wlt[:, H:H + A])
            + nt(oh_o1, wlt[:, H + A:]) + bloc_ref[...])
    sL = jnp.sum(oh_loc * rowL, axis=1)
    q = q + jnp.where(mL > 0.0, sL, _NEG)

    row2 = (nt(enc, w2t[:, :H]) + nt(oh_at, w2t[:, H:H + A])
            + nt(oh_o1, w2t[:, H + A:H + A + O])
            + nt(oh_loc, w2t[:, H + A + O:]) + bo2_ref[...])
    s2 = jnp.sum(oh_o2 * row2, axis=1)
    q = q + jnp.where(m2 > 0.0, s2, _NEG)

    out_ref[...] = q


def _tc_combine(enc, watt, w1t, wlt, w2t, at, o1, loc, o2, mA, m1, mL, m2,
                b_at, b_o1, b_loc, b_o2):
    B, H = enc.shape
    nb = B // _BLK

    def rows(i):
        return (i, 0)

    def full(i):
        return (0, 0)

    def fixed(a):
        return pl.BlockSpec(a.shape, full)

    def col(a):
        return pl.BlockSpec((_BLK, 1), rows)

    def mrow(a):
        return pl.BlockSpec((1, _BLK // 128, 128), lambda i: (i, 0, 0))

    return pl.pallas_call(
        _combine_body,
        grid=(nb,),
        in_specs=[
            pl.BlockSpec((_BLK, H), rows),
            fixed(watt), fixed(w1t), fixed(wlt), fixed(w2t),
            col(at), col(o1), col(loc), col(o2),
            mrow(mA), mrow(m1), mrow(mL), mrow(m2),
            fixed(b_at), fixed(b_o1), fixed(b_loc), fixed(b_o2),
        ],
        out_specs=pl.BlockSpec((_BLK,), lambda i: (i,)),
        out_shape=jax.ShapeDtypeStruct((B,), jnp.float32),
    )(enc, watt, w1t, wlt, w2t, at, o1, loc, o2, mA, m1, mL, m2,
      b_at, b_o1, b_loc, b_o2)


def kernel(encoded, action_types, object1, location, object2,
           action_type_masks, object1_masks, location_masks, object2_masks,
           W_at, b_at, W_o1, b_o1, W_loc, b_loc, W_o2, b_o2):
    B, H = encoded.shape
    A = action_type_masks.shape[1]
    O = object1_masks.shape[2]
    L = location_masks.shape[2]

    at = action_types[:, 0].astype(jnp.int32)
    o1i = object1[:, 0].astype(jnp.int32)
    loci = location[:, 0].astype(jnp.int32)
    o2i = object2[:, 0].astype(jnp.int32)

    tA = action_type_masks.transpose(1, 0)
    t1 = object1_masks.transpose(1, 2, 0).reshape(A * O, B)
    tL = location_masks.transpose(1, 2, 0).reshape(A * L, B)
    t2 = object2_masks.transpose(1, 2, 3, 0).reshape(A * O * O, B)

    jA = at
    j1 = at * O + o1i
    jL = at * L + loci
    j2 = (at * O + o1i) * O + o2i

    mA, m1, mL, m2 = _sc_gather_masks(tA, t1, tL, t2, jA, j1, jL, j2)

    nb = B // 128
    q = _tc_combine(
        encoded,
        W_at.transpose(1, 0), W_o1.transpose(1, 0), W_loc.transpose(1, 0),
        W_o2.transpose(1, 0),
        action_types.astype(jnp.int32), object1.astype(jnp.int32),
        location.astype(jnp.int32), object2.astype(jnp.int32),
        mA.reshape(B // _BLK, _BLK // 128, 128),
        m1.reshape(B // _BLK, _BLK // 128, 128),
        mL.reshape(B // _BLK, _BLK // 128, 128),
        m2.reshape(B // _BLK, _BLK // 128, 128),
        b_at.reshape(1, A), b_o1.reshape(1, O), b_loc.reshape(1, L),
        b_o2.reshape(1, O))
    return q[:, None]


# speedup vs baseline: 11.6002x; 1.0654x over previous
import functools

import jax
import jax.numpy as jnp
import numpy as np
from jax import lax
from jax.experimental import pallas as pl
from jax.experimental.pallas import tpu as pltpu
from jax.experimental.pallas import tpu_sc as plsc

_NEG = np.float32(-1e9)
_BLK = 256
_NT = (((1,), (1,)), ((), ()))


def _sc_gather_masks(tA, t1, tL, t2, ati, o1i, loci, o2i, O, L):
    B = ati.shape[0]
    info = plsc.get_sparse_core_info()
    nw = info.num_cores * info.num_subcores
    bw = B // nw
    mesh = plsc.VectorSubcoreMesh(core_axis_name="c", subcore_axis_name="s")
    f32 = jnp.float32

    @functools.partial(
        pl.kernel,
        mesh=mesh,
        out_type=tuple(jax.ShapeDtypeStruct((B,), f32) for _ in range(4)),
        scratch_types=[
            pltpu.VMEM((4, bw), jnp.int32),
            pltpu.VMEM((4, bw), jnp.int32),
            pltpu.VMEM((bw, 128), f32),
            pltpu.VMEM((bw, 128), f32),
            pltpu.VMEM((bw, 128), f32),
            pltpu.VMEM((bw, 128), f32),
            pltpu.VMEM((bw,), f32),
            pltpu.SemaphoreType.DMA,
        ],
    )
    def k(tA_h, t1_h, tL_h, t2_h, at_h, o1_h, loc_h, o2_h,
          oA_h, o1m_h, oL_h, o2m_h, iv, idx_v, bA_v, b1_v, bL_v, b2_v,
          out_v, sem):
        wid = lax.axis_index("s") * info.num_cores + lax.axis_index("c")
        base = wid * bw
        cb = (base // 128) * 128
        co = base - cb
        for i, h in enumerate((at_h, o1_h, loc_h, o2_h)):
            pltpu.sync_copy(h.at[pl.ds(base, bw)], iv.at[i])
        for g in range(bw // 16):
            sl = pl.ds(g * 16, 16)
            atv = iv[0, sl]
            o1v = iv[1, sl]
            locv = iv[2, sl]
            o2v = iv[3, sl]
            idx_v[0, sl] = atv
            idx_v[1, sl] = atv * O + o1v
            idx_v[2, sl] = atv * L + locv
            idx_v[3, sl] = (atv * O + o1v) * O + o2v
        copies = [
            pltpu.async_copy(t_h.at[idx_v.at[i], pl.ds(cb, 128)], b_v, sem)
            for i, (t_h, b_v) in enumerate(
                ((tA_h, bA_v), (t1_h, b1_v), (tL_h, bL_v), (t2_h, b2_v)))
        ]
        for c in copies:
            c.wait()
        it = lax.iota(jnp.int32, 16)
        for o_h, b_v in ((oA_h, bA_v), (o1m_h, b1_v), (oL_h, bL_v),
                         (o2m_h, b2_v)):
            for g in range(bw // 16):
                acc = jnp.zeros((16,), f32)
                for r in range(16):
                    row = b_v[g * 16 + r, pl.ds(co + g * 16, 16)]
                    acc = jnp.where(it == r, row, acc)
                out_v[pl.ds(g * 16, 16)] = acc
            pltpu.sync_copy(out_v, o_h.at[pl.ds(base, bw)])

    return k(tA, t1, tL, t2, ati, o1i, loci, o2i)


def _scores_body(enc_ref, watt_ref, w1t_ref, wlt_ref, w2t_ref,
                 at_ref, o1_ref, loc_ref, o2_ref,
                 bat_ref, bo1_ref, bloc_ref, bo2_ref,
                 oA_ref, o1_ref_out, oL_ref, o2_ref_out):
    f32 = jnp.float32
    H = enc_ref.shape[1]
    A = watt_ref.shape[0]
    O = w1t_ref.shape[0]
    n = enc_ref.shape[0]
    enc = enc_ref[...]

    def nt(x, y):
        return lax.dot_general(x, y, _NT, preferred_element_type=f32)

    at = at_ref[:, 0]
    o1 = o1_ref[:, 0]
    loc = loc_ref[:, 0]
    o2 = o2_ref[:, 0]
    ioA = lax.broadcasted_iota(jnp.int32, (n, A), 1)
    io64 = lax.broadcasted_iota(jnp.int32, (n, O), 1)
    oh_at = (ioA == at[:, None]).astype(f32)
    oh_o1 = (io64 == o1[:, None]).astype(f32)
    oh_loc = (io64 == loc[:, None]).astype(f32)
    oh_o2 = (io64 == o2[:, None]).astype(f32)

    w1t = w1t_ref[...]
    wlt = wlt_ref[...]
    w2t = w2t_ref[...]

    row_at = nt(enc, watt_ref[...]) + bat_ref[...]
    oA_ref[...] = jnp.sum(oh_at * row_at, axis=1)

    row1 = nt(enc, w1t[:, :H]) + nt(oh_at, w1t[:, H:]) + bo1_ref[...]
    o1_ref_out[...] = jnp.sum(oh_o1 * row1, axis=1)

    rowL = (nt(enc, wlt[:, :H]) + nt(oh_at, wlt[:, H:H + A])
            + nt(oh_o1, wlt[:, H + A:]) + bloc_ref[...])
    oL_ref[...] = jnp.sum(oh_loc * rowL, axis=1)

    row2 = (nt(enc, w2t[:, :H]) + nt(oh_at, w2t[:, H:H + A])
            + nt(oh_o1, w2t[:, H + A:H + A + O])
            + nt(oh_loc, w2t[:, H + A + O:]) + bo2_ref[...])
    o2_ref_out[...] = jnp.sum(oh_o2 * row2, axis=1)


def _tc_scores(enc, watt, w1t, wlt, w2t, at, o1, loc, o2,
               b_at, b_o1, b_loc, b_o2):
    B, H = enc.shape

    def rows(i):
        return (i, 0)

    def full(i):
        return (0, 0)

    def fixed(a):
        return pl.BlockSpec(a.shape, full)

    def col(a):
        return pl.BlockSpec((_BLK, 1), rows)

    ospec = pl.BlockSpec((_BLK,), lambda i: (i,))
    oshape = jax.ShapeDtypeStruct((B,), jnp.float32)
    return pl.pallas_call(
        _scores_body,
        grid=(B // _BLK,),
        in_specs=[
            pl.BlockSpec((_BLK, H), rows),
            fixed(watt), fixed(w1t), fixed(wlt), fixed(w2t),
            col(at), col(o1), col(loc), col(o2),
            fixed(b_at), fixed(b_o1), fixed(b_loc), fixed(b_o2),
        ],
        out_specs=(ospec, ospec, ospec, ospec),
        out_shape=(oshape, oshape, oshape, oshape),
    )(enc, watt, w1t, wlt, w2t, at, o1, loc, o2, b_at, b_o1, b_loc, b_o2)


def _combine_body(sA_ref, s1_ref, sL_ref, s2_ref,
                  mA_ref, m1_ref, mL_ref, m2_ref, out_ref):
    q = jnp.where(mA_ref[...] > 0.0, sA_ref[...], _NEG)
    q = q + jnp.where(m1_ref[...] > 0.0, s1_ref[...], _NEG)
    q = q + jnp.where(mL_ref[...] > 0.0, sL_ref[...], _NEG)
    q = q + jnp.where(m2_ref[...] > 0.0, s2_ref[...], _NEG)
    out_ref[...] = q


def _tc_combine(sA, s1, sL, s2, mA, m1, mL, m2):
    spec = pl.BlockSpec(sA.shape, lambda: (0, 0))
    return pl.pallas_call(
        _combine_body,
        in_specs=[spec] * 8,
        out_specs=spec,
        out_shape=jax.ShapeDtypeStruct(sA.shape, jnp.float32),
    )(sA, s1, sL, s2, mA, m1, mL, m2)


def kernel(encoded, action_types, object1, location, object2,
           action_type_masks, object1_masks, location_masks, object2_masks,
           W_at, b_at, W_o1, b_o1, W_loc, b_loc, W_o2, b_o2):
    B, H = encoded.shape
    A = action_type_masks.shape[1]
    O = object1_masks.shape[2]
    L = location_masks.shape[2]

    at = action_types[:, 0].astype(jnp.int32)
    o1i = object1[:, 0].astype(jnp.int32)
    loci = location[:, 0].astype(jnp.int32)
    o2i = object2[:, 0].astype(jnp.int32)

    tA = action_type_masks.transpose(1, 0)
    t1 = object1_masks.transpose(1, 2, 0).reshape(A * O, B)
    tL = location_masks.transpose(1, 2, 0).reshape(A * L, B)
    t2 = object2_masks.transpose(1, 2, 3, 0).reshape(A * O * O, B)

    mA, m1, mL, m2 = _sc_gather_masks(tA, t1, tL, t2, at, o1i, loci, o2i,
                                      O, L)

    sA, s1, sL, s2 = _tc_scores(
        encoded,
        W_at.transpose(1, 0), W_o1.transpose(1, 0), W_loc.transpose(1, 0),
        W_o2.transpose(1, 0),
        action_types.astype(jnp.int32), object1.astype(jnp.int32),
        location.astype(jnp.int32), object2.astype(jnp.int32),
        b_at.reshape(1, A), b_o1.reshape(1, O), b_loc.reshape(1, L),
        b_o2.reshape(1, O))

    t8 = (B // 128, 128)
    q = _tc_combine(sA.reshape(t8), s1.reshape(t8), sL.reshape(t8),
                    s2.reshape(t8), mA.reshape(t8), m1.reshape(t8),
                    mL.reshape(t8), m2.reshape(t8))
    return q.reshape(B, 1)


# speedup vs baseline: 12.8457x; 1.1074x over previous
import functools

import jax
import jax.numpy as jnp
import numpy as np
from jax import lax
from jax.experimental import pallas as pl
from jax.experimental.pallas import tpu as pltpu
from jax.experimental.pallas import tpu_sc as plsc

_NEG = np.float32(-1e9)
_BLK = 256
_NT = (((1,), (1,)), ((), ()))


def _sc_gather_masks(tA, t1, tL, t2, ati, o1i, loci, o2i, O, L):
    B = ati.shape[0]
    info = plsc.get_sparse_core_info()
    nw = info.num_cores * info.num_subcores
    bw = B // nw
    mesh = plsc.VectorSubcoreMesh(core_axis_name="c", subcore_axis_name="s")
    f32 = jnp.float32

    @functools.partial(
        pl.kernel,
        mesh=mesh,
        out_type=tuple(jax.ShapeDtypeStruct((B,), f32) for _ in range(4)),
        scratch_types=[
            pltpu.VMEM((4, bw), jnp.int32),
            pltpu.VMEM((4, bw), jnp.int32),
            pltpu.VMEM((bw, 128), f32),
            pltpu.VMEM((bw, 128), f32),
            pltpu.VMEM((bw, 128), f32),
            pltpu.VMEM((bw, 128), f32),
            pltpu.VMEM((bw,), f32),
            pltpu.SemaphoreType.DMA,
        ],
    )
    def k(tA_h, t1_h, tL_h, t2_h, at_h, o1_h, loc_h, o2_h,
          oA_h, o1m_h, oL_h, o2m_h, iv, idx_v, bA_v, b1_v, bL_v, b2_v,
          out_v, sem):
        wid = lax.axis_index("s") * info.num_cores + lax.axis_index("c")
        base = wid * bw
        cb = (base // 128) * 128
        co = base - cb
        for i, h in enumerate((at_h, o1_h, loc_h, o2_h)):
            pltpu.sync_copy(h.at[pl.ds(base, bw)], iv.at[i])
        for g in range(bw // 16):
            sl = pl.ds(g * 16, 16)
            atv = iv[0, sl]
            o1v = iv[1, sl]
            locv = iv[2, sl]
            o2v = iv[3, sl]
            idx_v[0, sl] = atv
            idx_v[1, sl] = atv * O + o1v
            idx_v[2, sl] = atv * L + locv
            idx_v[3, sl] = (atv * O + o1v) * O + o2v
        copies = [
            pltpu.async_copy(t_h.at[idx_v.at[i], pl.ds(cb, 128)], b_v, sem)
            for i, (t_h, b_v) in enumerate(
                ((tA_h, bA_v), (t1_h, b1_v), (tL_h, bL_v), (t2_h, b2_v)))
        ]
        for c in copies:
            c.wait()
        it = lax.iota(jnp.int32, 16)
        for o_h, b_v in ((oA_h, bA_v), (o1m_h, b1_v), (oL_h, bL_v),
                         (o2m_h, b2_v)):
            for g in range(bw // 16):
                acc = jnp.zeros((16,), f32)
                for r in range(16):
                    row = b_v[g * 16 + r, pl.ds(co + g * 16, 16)]
                    acc = jnp.where(it == r, row, acc)
                out_v[pl.ds(g * 16, 16)] = acc
            pltpu.sync_copy(out_v, o_h.at[pl.ds(base, bw)])

    return k(tA, t1, tL, t2, ati, o1i, loci, o2i)


def _scores_body(enc_ref, watt_ref, w1t_ref, wlt_ref, w2t_ref,
                 at_ref, o1_ref, loc_ref, o2_ref,
                 bat_ref, bo1_ref, bloc_ref, bo2_ref,
                 oA_ref, o1_ref_out, oL_ref, o2_ref_out):
    f32 = jnp.float32
    H = enc_ref.shape[1]
    A = watt_ref.shape[0]
    O = w1t_ref.shape[0]
    n = enc_ref.shape[0]
    enc = enc_ref[...]

    def nt(x, y):
        return lax.dot_general(x, y, _NT, preferred_element_type=f32)

    def to_col(ref):
        t = jnp.swapaxes(ref[0], 0, 1)
        return jnp.concatenate(
            [t[:, i:i + 1] for i in range(t.shape[1])], axis=0)

    at = to_col(at_ref)
    o1 = to_col(o1_ref)
    loc = to_col(loc_ref)
    o2 = to_col(o2_ref)
    ioA = lax.broadcasted_iota(jnp.int32, (n, A), 1)
    io64 = lax.broadcasted_iota(jnp.int32, (n, O), 1)
    oh_at = (ioA == at).astype(f32)
    oh_o1 = (io64 == o1).astype(f32)
    oh_loc = (io64 == loc).astype(f32)
    oh_o2 = (io64 == o2).astype(f32)

    w1t = w1t_ref[...]
    wlt = wlt_ref[...]
    w2t = w2t_ref[...]

    row_at = nt(enc, watt_ref[...]) + bat_ref[...]
    oA_ref[...] = jnp.sum(oh_at * row_at, axis=1)

    row1 = nt(enc, w1t[:, :H]) + nt(oh_at, w1t[:, H:]) + bo1_ref[...]
    o1_ref_out[...] = jnp.sum(oh_o1 * row1, axis=1)

    rowL = (nt(enc, wlt[:, :H]) + nt(oh_at, wlt[:, H:H + A])
            + nt(oh_o1, wlt[:, H + A:]) + bloc_ref[...])
    oL_ref[...] = jnp.sum(oh_loc * rowL, axis=1)

    row2 = (nt(enc, w2t[:, :H]) + nt(oh_at, w2t[:, H:H + A])
            + nt(oh_o1, w2t[:, H + A:H + A + O])
            + nt(oh_loc, w2t[:, H + A + O:]) + bo2_ref[...])
    o2_ref_out[...] = jnp.sum(oh_o2 * row2, axis=1)


def _tc_scores(enc, watt, w1t, wlt, w2t, at, o1, loc, o2,
               b_at, b_o1, b_loc, b_o2):
    B, H = enc.shape

    def rows(i):
        return (i, 0)

    def full(i):
        return (0, 0)

    def fixed(a):
        return pl.BlockSpec(a.shape, full)

    def mrow(a):
        return pl.BlockSpec((1, _BLK // 128, 128), lambda i: (i, 0, 0))

    ospec = pl.BlockSpec((_BLK,), lambda i: (i,))
    oshape = jax.ShapeDtypeStruct((B,), jnp.float32)
    return pl.pallas_call(
        _scores_body,
        grid=(B // _BLK,),
        in_specs=[
            pl.BlockSpec((_BLK, H), rows),
            fixed(watt), fixed(w1t), fixed(wlt), fixed(w2t),
            mrow(at), mrow(o1), mrow(loc), mrow(o2),
            fixed(b_at), fixed(b_o1), fixed(b_loc), fixed(b_o2),
        ],
        out_specs=(ospec, ospec, ospec, ospec),
        out_shape=(oshape, oshape, oshape, oshape),
    )(enc, watt, w1t, wlt, w2t, at, o1, loc, o2, b_at, b_o1, b_loc, b_o2)


def _combine_body(sA_ref, s1_ref, sL_ref, s2_ref,
                  mA_ref, m1_ref, mL_ref, m2_ref, out_ref):
    q = jnp.where(mA_ref[...] > 0.0, sA_ref[...], _NEG)
    q = q + jnp.where(m1_ref[...] > 0.0, s1_ref[...], _NEG)
    q = q + jnp.where(mL_ref[...] > 0.0, sL_ref[...], _NEG)
    q = q + jnp.where(m2_ref[...] > 0.0, s2_ref[...], _NEG)
    out_ref[...] = q


def _tc_combine(sA, s1, sL, s2, mA, m1, mL, m2):
    spec = pl.BlockSpec(sA.shape, lambda: (0, 0))
    return pl.pallas_call(
        _combine_body,
        in_specs=[spec] * 8,
        out_specs=spec,
        out_shape=jax.ShapeDtypeStruct(sA.shape, jnp.float32),
    )(sA, s1, sL, s2, mA, m1, mL, m2)


def kernel(encoded, action_types, object1, location, object2,
           action_type_masks, object1_masks, location_masks, object2_masks,
           W_at, b_at, W_o1, b_o1, W_loc, b_loc, W_o2, b_o2):
    B, H = encoded.shape
    A = action_type_masks.shape[1]
    O = object1_masks.shape[2]
    L = location_masks.shape[2]

    at = action_types[:, 0].astype(jnp.int32)
    o1i = object1[:, 0].astype(jnp.int32)
    loci = location[:, 0].astype(jnp.int32)
    o2i = object2[:, 0].astype(jnp.int32)

    tA = action_type_masks.transpose(1, 0)
    t1 = object1_masks.transpose(1, 2, 0).reshape(A * O, B)
    tL = location_masks.transpose(1, 2, 0).reshape(A * L, B)
    t2 = object2_masks.transpose(1, 2, 3, 0).reshape(A * O * O, B)

    mA, m1, mL, m2 = _sc_gather_masks(tA, t1, tL, t2, at, o1i, loci, o2i,
                                      O, L)

    i3 = (B // _BLK, _BLK // 128, 128)
    sA, s1, sL, s2 = _tc_scores(
        encoded,
        W_at.transpose(1, 0), W_o1.transpose(1, 0), W_loc.transpose(1, 0),
        W_o2.transpose(1, 0),
        at.reshape(i3), o1i.reshape(i3), loci.reshape(i3), o2i.reshape(i3),
        b_at.reshape(1, A), b_o1.reshape(1, O), b_loc.reshape(1, L),
        b_o2.reshape(1, O))

    t8 = (B // 128, 128)
    q = _tc_combine(sA.reshape(t8), s1.reshape(t8), sL.reshape(t8),
                    s2.reshape(t8), mA.reshape(t8), m1.reshape(t8),
                    mL.reshape(t8), m2.reshape(t8))
    return q.reshape(B, 1)


# speedup vs baseline: 12.9124x; 1.0052x over previous
import functools

import jax
import jax.numpy as jnp
import numpy as np
from jax import lax
from jax.experimental import pallas as pl
from jax.experimental.pallas import tpu as pltpu
from jax.experimental.pallas import tpu_sc as plsc

_NEG = np.float32(-1e9)
_BLK = 512
_NT = (((1,), (1,)), ((), ()))


def _sc_gather_masks(tA, t1, tL, t2, ati, o1i, loci, o2i, O, L):
    B = ati.shape[0]
    info = plsc.get_sparse_core_info()
    nw = info.num_cores * info.num_subcores
    bw = B // nw
    mesh = plsc.VectorSubcoreMesh(core_axis_name="c", subcore_axis_name="s")
    f32 = jnp.float32

    @functools.partial(
        pl.kernel,
        mesh=mesh,
        out_type=tuple(jax.ShapeDtypeStruct((B,), f32) for _ in range(4)),
        scratch_types=[
            pltpu.VMEM((4, bw), jnp.int32),
            pltpu.VMEM((4, bw), jnp.int32),
            pltpu.VMEM((bw, 128), f32),
            pltpu.VMEM((bw, 128), f32),
            pltpu.VMEM((bw, 128), f32),
            pltpu.VMEM((bw, 128), f32),
            pltpu.VMEM((bw,), f32),
            pltpu.SemaphoreType.DMA,
        ],
    )
    def k(tA_h, t1_h, tL_h, t2_h, at_h, o1_h, loc_h, o2_h,
          oA_h, o1m_h, oL_h, o2m_h, iv, idx_v, bA_v, b1_v, bL_v, b2_v,
          out_v, sem):
        wid = lax.axis_index("s") * info.num_cores + lax.axis_index("c")
        base = wid * bw
        cb = (base // 128) * 128
        co = base - cb
        for i, h in enumerate((at_h, o1_h, loc_h, o2_h)):
            pltpu.sync_copy(h.at[pl.ds(base, bw)], iv.at[i])
        for g in range(bw // 16):
            sl = pl.ds(g * 16, 16)
            atv = iv[0, sl]
            o1v = iv[1, sl]
            locv = iv[2, sl]
            o2v = iv[3, sl]
            idx_v[0, sl] = atv
            idx_v[1, sl] = atv * O + o1v
            idx_v[2, sl] = atv * L + locv
            idx_v[3, sl] = (atv * O + o1v) * O + o2v
        copies = [
            pltpu.async_copy(t_h.at[idx_v.at[i], pl.ds(cb, 128)], b_v, sem)
            for i, (t_h, b_v) in enumerate(
                ((tA_h, bA_v), (t1_h, b1_v), (tL_h, bL_v), (t2_h, b2_v)))
        ]
        for c in copies:
            c.wait()
        it = lax.iota(jnp.int32, 16)
        for o_h, b_v in ((oA_h, bA_v), (o1m_h, b1_v), (oL_h, bL_v),
                         (o2m_h, b2_v)):
            for g in range(bw // 16):
                acc = jnp.zeros((16,), f32)
                for r in range(16):
                    row = b_v[g * 16 + r, pl.ds(co + g * 16, 16)]
                    acc = jnp.where(it == r, row, acc)
                out_v[pl.ds(g * 16, 16)] = acc
            pltpu.sync_copy(out_v, o_h.at[pl.ds(base, bw)])

    return k(tA, t1, tL, t2, ati, o1i, loci, o2i)


def _scores_body(enc_ref, watt_ref, w1t_ref, wlt_ref, w2t_ref,
                 at_ref, o1_ref, loc_ref, o2_ref,
                 bat_ref, bo1_ref, bloc_ref, bo2_ref,
                 oA_ref, o1_ref_out, oL_ref, o2_ref_out):
    f32 = jnp.float32
    H = enc_ref.shape[1]
    A = watt_ref.shape[0]
    O = w1t_ref.shape[0]
    n = enc_ref.shape[0]
    enc = enc_ref[...]

    def nt(x, y):
        return lax.dot_general(x, y, _NT, preferred_element_type=f32)

    def to_col(ref):
        t = jnp.swapaxes(ref[0], 0, 1)
        return jnp.concatenate(
            [t[:, i:i + 1] for i in range(t.shape[1])], axis=0)

    at = to_col(at_ref)
    o1 = to_col(o1_ref)
    loc = to_col(loc_ref)
    o2 = to_col(o2_ref)
    ioA = lax.broadcasted_iota(jnp.int32, (n, A), 1)
    io64 = lax.broadcasted_iota(jnp.int32, (n, O), 1)
    oh_at = (ioA == at).astype(f32)
    oh_o1 = (io64 == o1).astype(f32)
    oh_loc = (io64 == loc).astype(f32)
    oh_o2 = (io64 == o2).astype(f32)

    w1t = w1t_ref[...]
    wlt = wlt_ref[...]
    w2t = w2t_ref[...]

    row_at = nt(enc, watt_ref[...]) + bat_ref[...]
    oA_ref[...] = jnp.sum(oh_at * row_at, axis=1)

    row1 = nt(enc, w1t[:, :H]) + nt(oh_at, w1t[:, H:]) + bo1_ref[...]
    o1_ref_out[...] = jnp.sum(oh_o1 * row1, axis=1)

    rowL = (nt(enc, wlt[:, :H]) + nt(oh_at, wlt[:, H:H + A])
            + nt(oh_o1, wlt[:, H + A:]) + bloc_ref[...])
    oL_ref[...] = jnp.sum(oh_loc * rowL, axis=1)

    row2 = (nt(enc, w2t[:, :H]) + nt(oh_at, w2t[:, H:H + A])
            + nt(oh_o1, w2t[:, H + A:H + A + O])
            + nt(oh_loc, w2t[:, H + A + O:]) + bo2_ref[...])
    o2_ref_out[...] = jnp.sum(oh_o2 * row2, axis=1)


def _tc_scores(enc, watt, w1t, wlt, w2t, at, o1, loc, o2,
               b_at, b_o1, b_loc, b_o2):
    B, H = enc.shape

    def rows(i):
        return (i, 0)

    def full(i):
        return (0, 0)

    def fixed(a):
        return pl.BlockSpec(a.shape, full)

    def mrow(a):
        return pl.BlockSpec((1, _BLK // 128, 128), lambda i: (i, 0, 0))

    ospec = pl.BlockSpec((_BLK,), lambda i: (i,))
    oshape = jax.ShapeDtypeStruct((B,), jnp.float32)
    return pl.pallas_call(
        _scores_body,
        grid=(B // _BLK,),
        in_specs=[
            pl.BlockSpec((_BLK, H), rows),
            fixed(watt), fixed(w1t), fixed(wlt), fixed(w2t),
            mrow(at), mrow(o1), mrow(loc), mrow(o2),
            fixed(b_at), fixed(b_o1), fixed(b_loc), fixed(b_o2),
        ],
        out_specs=(ospec, ospec, ospec, ospec),
        out_shape=(oshape, oshape, oshape, oshape),
    )(enc, watt, w1t, wlt, w2t, at, o1, loc, o2, b_at, b_o1, b_loc, b_o2)


def _combine_body(sA_ref, s1_ref, sL_ref, s2_ref,
                  mA_ref, m1_ref, mL_ref, m2_ref, out_ref):
    q = jnp.where(mA_ref[...] > 0.0, sA_ref[...], _NEG)
    q = q + jnp.where(m1_ref[...] > 0.0, s1_ref[...], _NEG)
    q = q + jnp.where(mL_ref[...] > 0.0, sL_ref[...], _NEG)
    q = q + jnp.where(m2_ref[...] > 0.0, s2_ref[...], _NEG)
    out_ref[...] = q


def _tc_combine(sA, s1, sL, s2, mA, m1, mL, m2):
    spec = pl.BlockSpec(sA.shape, lambda: (0, 0))
    return pl.pallas_call(
        _combine_body,
        in_specs=[spec] * 8,
        out_specs=spec,
        out_shape=jax.ShapeDtypeStruct(sA.shape, jnp.float32),
    )(sA, s1, sL, s2, mA, m1, mL, m2)


def kernel(encoded, action_types, object1, location, object2,
           action_type_masks, object1_masks, location_masks, object2_masks,
           W_at, b_at, W_o1, b_o1, W_loc, b_loc, W_o2, b_o2):
    B, H = encoded.shape
    A = action_type_masks.shape[1]
    O = object1_masks.shape[2]
    L = location_masks.shape[2]

    at = action_types[:, 0].astype(jnp.int32)
    o1i = object1[:, 0].astype(jnp.int32)
    loci = location[:, 0].astype(jnp.int32)
    o2i = object2[:, 0].astype(jnp.int32)

    tA = action_type_masks.transpose(1, 0)
    t1 = object1_masks.transpose(1, 2, 0).reshape(A * O, B)
    tL = location_masks.transpose(1, 2, 0).reshape(A * L, B)
    t2 = object2_masks.transpose(1, 2, 3, 0).reshape(A * O * O, B)

    mA, m1, mL, m2 = _sc_gather_masks(tA, t1, tL, t2, at, o1i, loci, o2i,
                                      O, L)

    i3 = (B // _BLK, _BLK // 128, 128)
    sA, s1, sL, s2 = _tc_scores(
        encoded,
        W_at.transpose(1, 0), W_o1.transpose(1, 0), W_loc.transpose(1, 0),
        W_o2.transpose(1, 0),
        at.reshape(i3), o1i.reshape(i3), loci.reshape(i3), o2i.reshape(i3),
        b_at.reshape(1, A), b_o1.reshape(1, O), b_loc.reshape(1, L),
        b_o2.reshape(1, O))

    t8 = (B // 128, 128)
    q = _tc_combine(sA.reshape(t8), s1.reshape(t8), sL.reshape(t8),
                    s2.reshape(t8), mA.reshape(t8), m1.reshape(t8),
                    mL.reshape(t8), m2.reshape(t8))
    return q.reshape(B, 1)
